# baseline (speedup 1.0000x reference)
import functools

import jax
import jax.numpy as jnp
from jax import lax
from jax.experimental import pallas as pl
from jax.experimental.pallas import tpu as pltpu
from jax.experimental.pallas import tpu_sc as plsc

_N = 10000
_E = 320000
_NP = 10240
_NPX = _NP + 64
_INDEX = 123
_NC = 2
_NS = 16
_NW = _NC * _NS
_EPW = _E // _NW
_ECH = 400
_NCHK = _EPW // _ECH
_CAP = 10048
_ACH = 64
_RPT = _NP // _NW
_BR = 512


def _mesh():
    return plsc.VectorSubcoreMesh(core_axis_name="c", subcore_axis_name="s")


def _iota16():
    return lax.iota(jnp.int32, 16)


def _sc_prep(src, dst, ewp):
    @functools.partial(
        pl.kernel,
        out_type=(
            jax.ShapeDtypeStruct((_E,), jnp.float32),
            jax.ShapeDtypeStruct((_NC * _NP,), jnp.float32),
            jax.ShapeDtypeStruct((_NC * _NPX,), jnp.int32),
        ),
        mesh=_mesh(),
        compiler_params=pltpu.CompilerParams(needs_layout_passes=False),
        scratch_types=(
            pltpu.VMEM((_ECH,), jnp.int32),
            pltpu.VMEM((_ECH,), jnp.int32),
            pltpu.VMEM((_ECH,), jnp.float32),
            pltpu.VMEM((_ECH,), jnp.float32),
            pltpu.VMEM((_ECH,), jnp.int32),
            pltpu.VMEM((_ECH,), jnp.int32),
            pltpu.VMEM((_RPT,), jnp.float32),
            pltpu.VMEM((_RPT,), jnp.int32),
            pltpu.VMEM((16,), jnp.int32),
            pltpu.VMEM_SHARED((_NP,), jnp.float32),
        ),
    )
    def k(src_h, dst_h, ewp_h, sig_h, degp_h, needa_h,
          srcb, dstb, ewb, sigb, idxb, oneb, zf, zi, sib, spdeg):
        c = lax.axis_index("c")
        s = lax.axis_index("s")
        wid = s * _NC + c
        iota = _iota16()

        def initz(i, _):
            sl = pl.ds(i * 16, 16)
            zf[sl] = jnp.zeros((16,), jnp.float32)
            zi[sl] = jnp.zeros((16,), jnp.int32)
            return 0
        lax.fori_loop(0, _RPT // 16, initz, 0)

        def init1(i, _):
            oneb[pl.ds(i * 16, 16)] = jnp.ones((16,), jnp.int32)
            return 0
        lax.fori_loop(0, _ECH // 16, init1, 0)

        pltpu.sync_copy(zf, spdeg.at[pl.ds(s * _RPT, _RPT)])
        pltpu.sync_copy(zi, needa_h.at[pl.ds(c * _NPX + s * _RPT, _RPT)])
        plsc.subcore_barrier()

        def chunk(kk, _):
            base = wid * _EPW + kk * _ECH
            pltpu.sync_copy(src_h.at[pl.ds(base, _ECH)], srcb)
            pltpu.sync_copy(dst_h.at[pl.ds(base, _ECH)], dstb)
            pltpu.sync_copy(ewp_h.at[pl.ds(base, _ECH)], ewb)

            def inner(j, _2):
                sl = pl.ds(j * 16, 16)
                s16 = srcb[sl]
                d16 = dstb[sl]
                e16 = ewb[sl]
                sigb[sl] = 1.0 / (1.0 + jnp.exp(-e16))
                idxb[sl] = c * _NPX + jnp.where(d16 == _INDEX, s16, _NP)
                return 0
            lax.fori_loop(0, _ECH // 16, inner, 0)
            pltpu.sync_copy(sigb, sig_h.at[pl.ds(base, _ECH)])
            pltpu.sync_copy(sigb, spdeg.at[dstb], add=True)
            pltpu.sync_copy(oneb, needa_h.at[idxb])
            return 0
        lax.fori_loop(0, _NCHK, chunk, 0)

        @pl.when(s == 0)
        def _set_index_flag():
            sib[pl.ds(0, 16)] = c * _NPX + jnp.where(iota == 0, _INDEX, _NP)
            pltpu.sync_copy(oneb.at[pl.ds(0, 16)], needa_h.at[sib])

        plsc.subcore_barrier()
        pltpu.sync_copy(spdeg.at[pl.ds(s * _RPT, _RPT)], zf)
        pltpu.sync_copy(zf, degp_h.at[pl.ds(c * _NP + s * _RPT, _RPT)])

    return k(src, dst, ewp)


def _tc_dinv(degp):
    def body(d_ref, o_ref):
        o_ref[...] = lax.rsqrt(d_ref[0] + d_ref[1] + 1.0)

    return pl.pallas_call(
        body,
        grid=(_NP // _BR,),
        in_specs=[pl.BlockSpec((_NC, _BR), lambda i: (0, i))],
        out_specs=pl.BlockSpec((_BR,), lambda i: (i,)),
        out_shape=jax.ShapeDtypeStruct((_NP,), jnp.float32),
    )(degp.reshape(_NC, _NP))


def _sc_flag2(src, dst, needa):
    @functools.partial(
        pl.kernel,
        out_type=jax.ShapeDtypeStruct((_NC * _NPX,), jnp.int32),
        mesh=_mesh(),
        compiler_params=pltpu.CompilerParams(needs_layout_passes=False),
        scratch_types=(
            pltpu.VMEM((_NP,), jnp.int32),
            pltpu.VMEM((_NP,), jnp.int32),
            pltpu.VMEM((_ECH,), jnp.int32),
            pltpu.VMEM((_ECH,), jnp.int32),
            pltpu.VMEM((_ECH,), jnp.int32),
            pltpu.VMEM((_ECH,), jnp.int32),
        ),
    )
    def k(src_h, dst_h, needa_h, needb_h, fA, tmp, srcb, dstb, idxb, oneb):
        c = lax.axis_index("c")
        s = lax.axis_index("s")
        wid = s * _NC + c
        pltpu.sync_copy(needa_h.at[pl.ds(0, _NP)], fA)
        pltpu.sync_copy(needa_h.at[pl.ds(_NPX, _NP)], tmp)

        def mrg(i, _):
            sl = pl.ds(i * 16, 16)
            fA[sl] = fA[sl] | tmp[sl]
            return 0
        lax.fori_loop(0, _NP // 16, mrg, 0)

        def init1(i, _):
            oneb[pl.ds(i * 16, 16)] = jnp.ones((16,), jnp.int32)
            return 0
        lax.fori_loop(0, _ECH // 16, init1, 0)

        pltpu.sync_copy(fA.at[pl.ds(s * _RPT, _RPT)],
                        needb_h.at[pl.ds(c * _NPX + s * _RPT, _RPT)])
        plsc.subcore_barrier()

        def chunk(kk, _):
            base = wid * _EPW + kk * _ECH
            pltpu.sync_copy(src_h.at[pl.ds(base, _ECH)], srcb)
            pltpu.sync_copy(dst_h.at[pl.ds(base, _ECH)], dstb)

            def inner(j, _2):
                sl = pl.ds(j * 16, 16)
                d16 = dstb[sl]
                s16 = srcb[sl]
                f16 = plsc.load_gather(fA, [d16])
                idxb[sl] = c * _NPX + jnp.where(f16 > 0, s16, _NP)
                return 0
            lax.fori_loop(0, _ECH // 16, inner, 0)
            pltpu.sync_copy(oneb, needb_h.at[idxb])
            return 0
        lax.fori_loop(0, _NCHK, chunk, 0)

    return k(src, dst, needa)


def _sc_compact(src, dst, sig, dinv, needa, needb):
    @functools.partial(
        pl.kernel,
        out_type=(
            jax.ShapeDtypeStruct((3 * _NW * _CAP,), jnp.int32),
            jax.ShapeDtypeStruct((3 * _NW * _CAP,), jnp.int32),
            jax.ShapeDtypeStruct((3 * _NW * _CAP,), jnp.float32),
            jax.ShapeDtypeStruct((_NW * 16,), jnp.int32),
        ),
        mesh=_mesh(),
        compiler_params=pltpu.CompilerParams(needs_layout_passes=False),
        scratch_types=(
            pltpu.VMEM((_NP,), jnp.float32),
            pltpu.VMEM((_NP,), jnp.int32),
            pltpu.VMEM((_NP,), jnp.int32),
            pltpu.VMEM((_ECH,), jnp.int32),
            pltpu.VMEM((_ECH,), jnp.int32),
            pltpu.VMEM((_ECH,), jnp.float32),
            pltpu.VMEM((_CAP,), jnp.int32),
            pltpu.VMEM((_CAP,), jnp.int32),
            pltpu.VMEM((_CAP,), jnp.float32),
            pltpu.VMEM((_CAP,), jnp.int32),
            pltpu.VMEM((_CAP,), jnp.int32),
            pltpu.VMEM((_CAP,), jnp.float32),
            pltpu.VMEM((_CAP,), jnp.int32),
            pltpu.VMEM((_CAP,), jnp.int32),
            pltpu.VMEM((_CAP,), jnp.float32),
            pltpu.VMEM((16,), jnp.int32),
        ),
    )
    def k(src_h, dst_h, sig_h, dinv_h, needa_h, needb_h,
          srcc_h, dstc_h, normc_h, cnt_h,
          dinvv, fB, fA, srcb, dstb, sgb,
          s1s, s1d, s1n, s2s, s2d, s2n, s3s, s3d, s3n, cntb):
        c = lax.axis_index("c")
        s = lax.axis_index("s")
        wid = s * _NC + c
        iota = _iota16()

        pltpu.sync_copy(dinv_h, dinvv)
        pltpu.sync_copy(needa_h.at[pl.ds(0, _NP)], fA)
        pltpu.sync_copy(needb_h.at[pl.ds(0, _NP)], fB)

        def mrg(i, _):
            base = i * 320
            pltpu.sync_copy(needa_h.at[pl.ds(_NPX + base, 320)],
                            srcb.at[pl.ds(0, 320)])
            pltpu.sync_copy(needb_h.at[pl.ds(_NPX + base, 320)],
                            dstb.at[pl.ds(0, 320)])

            def m2(j, _2):
                sl = pl.ds(base + j * 16, 16)
                sb = pl.ds(j * 16, 16)
                fA[sl] = fA[sl] | srcb[sb]
                fB[sl] = fB[sl] | dstb[sb]
                return 0
            lax.fori_loop(0, 20, m2, 0)
            return 0
        lax.fori_loop(0, _NP // 320, mrg, 0)

        def zs(i, _):
            sl = pl.ds(i * 16, 16)
            zi = jnp.zeros((16,), jnp.int32)
            zft = jnp.zeros((16,), jnp.float32)
            s1s[sl] = zi; s1d[sl] = zi; s1n[sl] = zft
            s2s[sl] = zi; s2d[sl] = zi; s2n[sl] = zft
            s3s[sl] = zi; s3d[sl] = zi; s3n[sl] = zft
            return 0
        lax.fori_loop(0, _CAP // 16, zs, 0)

        def chunk(kk, carry):
            base = wid * _EPW + kk * _ECH
            pltpu.sync_copy(src_h.at[pl.ds(base, _ECH)], srcb)
            pltpu.sync_copy(dst_h.at[pl.ds(base, _ECH)], dstb)
            pltpu.sync_copy(sig_h.at[pl.ds(base, _ECH)], sgb)

            def inner(j, cr):
                c1, c2, c3 = cr
                sl = pl.ds(j * 16, 16)
                s16 = srcb[sl]
                d16 = dstb[sl]
                g16 = sgb[sl]
                fb16 = plsc.load_gather(fB, [d16])
                fa16 = plsc.load_gather(fA, [d16])
                n16 = plsc.load_gather(dinvv, [s16]) * g16 * \
                    plsc.load_gather(dinvv, [d16])
                m1 = fb16 > 0
                m2m = fa16 > 0
                m3 = d16 == _INDEX
                plsc.store_compressed(s1s.at[pl.ds(c1, 16)], s16, mask=m1)
                plsc.store_compressed(s1d.at[pl.ds(c1, 16)], d16, mask=m1)
                plsc.store_compressed(s1n.at[pl.ds(c1, 16)], n16, mask=m1)
                c1 = c1 + jnp.max(plsc.all_reduce_population_count(m1))
                plsc.store_compressed(s2s.at[pl.ds(c2, 16)], s16, mask=m2m)
                plsc.store_compressed(s2d.at[pl.ds(c2, 16)], d16, mask=m2m)
                plsc.store_compressed(s2n.at[pl.ds(c2, 16)], n16, mask=m2m)
                c2 = c2 + jnp.max(plsc.all_reduce_population_count(m2m))
                plsc.store_compressed(s3s.at[pl.ds(c3, 16)], s16, mask=m3)
                plsc.store_compressed(s3d.at[pl.ds(c3, 16)], d16, mask=m3)
                plsc.store_compressed(s3n.at[pl.ds(c3, 16)], n16, mask=m3)
                c3 = c3 + jnp.max(plsc.all_reduce_population_count(m3))
                return (c1, c2, c3)
            return lax.fori_loop(0, _ECH // 16, inner, carry)

        zero = jnp.zeros((), jnp.int32)
        c1, c2, c3 = lax.fori_loop(0, _NCHK, chunk, (zero, zero, zero))

        @pl.when(wid == 0)
        def _selfloop():
            dv = plsc.load_gather(dinvv, [jnp.where(iota == 0, _INDEX, 0)])
            s3s[pl.ds(c3, 16)] = jnp.where(iota == 0, _INDEX, 0)
            s3d[pl.ds(c3, 16)] = jnp.where(iota == 0, _INDEX, 0)
            s3n[pl.ds(c3, 16)] = jnp.where(iota == 0, dv * dv, 0.0)

        c3 = c3 + jnp.where(wid == 0, 1, 0)

        p1 = ((c1 + _ACH - 1) // _ACH) * _ACH
        p2 = ((c2 + _ACH - 1) // _ACH) * _ACH
        p3 = ((c3 + _ACH - 1) // _ACH) * _ACH
        cntb[pl.ds(0, 16)] = (jnp.where(iota == 0, p1, 0)
                              + jnp.where(iota == 1, p2, 0)
                              + jnp.where(iota == 2, p3, 0))
        pltpu.sync_copy(cntb, cnt_h.at[pl.ds(wid * 16, 16)])

        pltpu.sync_copy(s1s, srcc_h.at[pl.ds((0 * _NW + wid) * _CAP, _CAP)])
        pltpu.sync_copy(s1d, dstc_h.at[pl.ds((0 * _NW + wid) * _CAP, _CAP)])
        pltpu.sync_copy(s1n, normc_h.at[pl.ds((0 * _NW + wid) * _CAP, _CAP)])
        pltpu.sync_copy(s2s, srcc_h.at[pl.ds((1 * _NW + wid) * _CAP, _CAP)])
        pltpu.sync_copy(s2d, dstc_h.at[pl.ds((1 * _NW + wid) * _CAP, _CAP)])
        pltpu.sync_copy(s2n, normc_h.at[pl.ds((1 * _NW + wid) * _CAP, _CAP)])
        pltpu.sync_copy(s3s, srcc_h.at[pl.ds((2 * _NW + wid) * _CAP, _CAP)])
        pltpu.sync_copy(s3d, dstc_h.at[pl.ds((2 * _NW + wid) * _CAP, _CAP)])
        pltpu.sync_copy(s3n, normc_h.at[pl.ds((2 * _NW + wid) * _CAP, _CAP)])

    return k(src, dst, sig, dinv, needa, needb)


def _sc_agg(feat, srcc, dstc, normc, counts, lane):
    @functools.partial(
        pl.kernel,
        out_type=jax.ShapeDtypeStruct((_NC * _NP, 128), jnp.float32),
        mesh=_mesh(),
        compiler_params=pltpu.CompilerParams(needs_layout_passes=False),
        scratch_types=(
            pltpu.VMEM((_ACH,), jnp.int32),
            pltpu.VMEM((_ACH,), jnp.int32),
            pltpu.VMEM((_ACH,), jnp.float32),
            pltpu.VMEM((16,), jnp.int32),
            pltpu.VMEM((_ACH, 128), jnp.float32),
            pltpu.VMEM_SHARED((_NP, 128), jnp.float32),
            pltpu.SemaphoreType.DMA,
        ),
    )
    def k(feat_h, srcc_h, dstc_h, normc_h, cnt_h, aggp_h,
          sbuf, dbuf, nbuf, cntb, rows, spagg, sem):
        c = lax.axis_index("c")
        s = lax.axis_index("s")
        wid = s * _NC + c
        iota = _iota16()

        for i in range(_ACH):
            for q in range(8):
                rows[i, pl.ds(q * 16, 16)] = jnp.zeros((16,), jnp.float32)

        def zs(i, _):
            pltpu.sync_copy(
                rows, spagg.at[pl.ds(s * _RPT + i * _ACH, _ACH), :])
            return 0
        lax.fori_loop(0, _RPT // _ACH, zs, 0)
        plsc.subcore_barrier()

        pltpu.sync_copy(cnt_h.at[pl.ds(wid * 16, 16)], cntb)
        cnt = jnp.max(jnp.where(iota == lane, cntb[pl.ds(0, 16)], 0))
        lbase = (lane * _NW + wid) * _CAP

        def chunk(j, _):
            b = lbase + j * _ACH
            pltpu.sync_copy(srcc_h.at[pl.ds(b, _ACH)], sbuf)
            pltpu.sync_copy(dstc_h.at[pl.ds(b, _ACH)], dbuf)
            pltpu.sync_copy(normc_h.at[pl.ds(b, _ACH)], nbuf)
            pltpu.async_copy(feat_h.at[sbuf], rows, sem).wait()
            for g in range(4):
                n16 = nbuf[pl.ds(g * 16, 16)]
                for i in range(16):
                    sc = jnp.sum(jnp.where(iota == i, n16, 0.0))
                    for q in range(8):
                        sl = pl.ds(q * 16, 16)
                        rows[g * 16 + i, sl] = rows[g * 16 + i, sl] * sc
            pltpu.sync_copy(rows, spagg.at[dbuf], add=True)
            return 0
        lax.fori_loop(0, cnt // _ACH, chunk, 0)
        plsc.subcore_barrier()

        def ex(i, _):
            pltpu.sync_copy(
                spagg.at[pl.ds(s * _RPT + i * _ACH, _ACH), :], rows)
            pltpu.sync_copy(
                rows, aggp_h.at[pl.ds(c * _NP + s * _RPT + i * _ACH, _ACH), :])
            return 0
        lax.fori_loop(0, _RPT // _ACH, ex, 0)

    return k(feat, srcc, dstc, normc, counts)


def _tc_layer(aggp, dinv, feat, W, b):
    def body(a_ref, dv_ref, f_ref, w_ref, b_ref, o_ref):
        dv = dv_ref[...]
        a = a_ref[0] + a_ref[1] + (dv * dv)[:, None] * f_ref[...]
        o_ref[...] = jnp.maximum(
            jnp.dot(a, w_ref[...], preferred_element_type=jnp.float32)
            + b_ref[...], 0.0)

    return pl.pallas_call(
        body,
        grid=(_NP // _BR,),
        in_specs=[
            pl.BlockSpec((_NC, _BR, 128), lambda i: (0, i, 0)),
            pl.BlockSpec((_BR,), lambda i: (i,)),
            pl.BlockSpec((_BR, 128), lambda i: (i, 0)),
            pl.BlockSpec((128, 128), lambda i: (0, 0)),
            pl.BlockSpec((1, 128), lambda i: (0, 0)),
        ],
        out_specs=pl.BlockSpec((_BR, 128), lambda i: (i, 0)),
        out_shape=jax.ShapeDtypeStruct((_NP, 128), jnp.float32),
    )(aggp.reshape(_NC, _NP, 128), dinv, feat, W, b.reshape(1, -1))


def _sc_agg3(feat, srcc, normc, counts):
    @functools.partial(
        pl.kernel,
        out_type=jax.ShapeDtypeStruct((_NC, 128), jnp.float32),
        mesh=_mesh(),
        compiler_params=pltpu.CompilerParams(needs_layout_passes=False),
        scratch_types=(
            pltpu.VMEM((_ACH,), jnp.int32),
            pltpu.VMEM((_ACH,), jnp.float32),
            pltpu.VMEM((16,), jnp.int32),
            pltpu.VMEM((_ACH, 128), jnp.float32),
            pltpu.VMEM((_NS, 128), jnp.float32),
            pltpu.VMEM_SHARED((_NS, 128), jnp.float32),
            pltpu.SemaphoreType.DMA,
        ),
    )
    def k(feat_h, srcc_h, normc_h, cnt_h, out_h,
          sbuf, nbuf, cntb, rows, accb, sp3, sem):
        c = lax.axis_index("c")
        s = lax.axis_index("s")
        wid = s * _NC + c
        iota = _iota16()

        pltpu.sync_copy(cnt_h.at[pl.ds(wid * 16, 16)], cntb)
        cnt = jnp.max(jnp.where(iota == 2, cntb[pl.ds(0, 16)], 0))
        lbase = (2 * _NW + wid) * _CAP

        acc0 = tuple(jnp.zeros((16,), jnp.float32) for _ in range(8))

        def chunk(j, acc):
            b = lbase + j * _ACH
            pltpu.sync_copy(srcc_h.at[pl.ds(b, _ACH)], sbuf)
            pltpu.sync_copy(normc_h.at[pl.ds(b, _ACH)], nbuf)
            pltpu.async_copy(feat_h.at[sbuf], rows, sem).wait()
            acc = list(acc)
            for g in range(4):
                n16 = nbuf[pl.ds(g * 16, 16)]
                for i in range(16):
                    sc = jnp.sum(jnp.where(iota == i, n16, 0.0))
                    for q in range(8):
                        acc[q] = acc[q] + rows[g * 16 + i,
                                               pl.ds(q * 16, 16)] * sc
            return tuple(acc)
        acc = lax.fori_loop(0, cnt // _ACH, chunk, acc0)

        for q in range(8):
            accb[0, pl.ds(q * 16, 16)] = acc[q]
        pltpu.sync_copy(accb.at[pl.ds(0, 1), :], sp3.at[pl.ds(s, 1), :])
        plsc.subcore_barrier()

        @pl.when(s == 0)
        def _reduce():
            pltpu.sync_copy(sp3, accb)
            for q in range(8):
                sl = pl.ds(q * 16, 16)
                v = accb[0, sl]
                for r in range(1, _NS):
                    v = v + accb[r, sl]
                rows[0, sl] = v
            pltpu.sync_copy(rows.at[pl.ds(0, 1), :],
                            out_h.at[pl.ds(c, 1), :])

    return k(feat, srcc, normc, counts)


def _tc_final(agg3, W3p, b3p):
    def body(a_ref, w_ref, b_ref, o_ref):
        a = a_ref[pl.ds(0, 1), :] + a_ref[pl.ds(1, 1), :]
        z = jnp.dot(a, w_ref[...], preferred_element_type=jnp.float32) \
            + b_ref[...]
        m = jnp.max(z)
        lse = jnp.log(jnp.sum(jnp.exp(z - m))) + m
        o_ref[...] = z - lse

    return pl.pallas_call(
        body,
        grid=(1,),
        in_specs=[
            pl.BlockSpec((_NC, 128), lambda i: (0, 0)),
            pl.BlockSpec((128, 128), lambda i: (0, 0)),
            pl.BlockSpec((1, 128), lambda i: (0, 0)),
        ],
        out_specs=pl.BlockSpec((1, 128), lambda i: (0, 0)),
        out_shape=jax.ShapeDtypeStruct((1, 128), jnp.float32),
    )(agg3, W3p, b3p.reshape(1, -1))


def kernel(x, edge_index, edge_weight_params, W1, b1, W2, b2, W3, b3):
    src = edge_index[0].astype(jnp.int32)
    dst = edge_index[1].astype(jnp.int32)
    xp = jnp.pad(x, ((0, _NP - _N), (0, 0)))

    sig, degp, needa = _sc_prep(src, dst, edge_weight_params)
    dinv = _tc_dinv(degp)
    needb = _sc_flag2(src, dst, needa)
    srcc, dstc, normc, counts = _sc_compact(src, dst, sig, dinv, needa, needb)
    aggp1 = _sc_agg(xp, srcc, dstc, normc, counts, 0)
    h1 = _tc_layer(aggp1, dinv, xp, W1, b1)
    aggp2 = _sc_agg(h1, srcc, dstc, normc, counts, 1)
    h2 = _tc_layer(aggp2, dinv, h1, W2, b2)
    agg3 = _sc_agg3(h2, srcc, normc, counts)
    W3p = jnp.pad(W3, ((0, 0), (0, 128 - 16)))
    b3p = jnp.pad(b3, (0, 128 - 16), constant_values=-1e30)
    out = _tc_final(agg3, W3p, b3p)
    return out[0, :16]


# speedup vs baseline: 115.7688x; 115.7688x over previous
import functools

import jax
import jax.numpy as jnp
from jax import lax
from jax.experimental import pallas as pl
from jax.experimental.pallas import tpu as pltpu
from jax.experimental.pallas import tpu_sc as plsc

_N = 10000
_E = 320000
_NP = 10240
_NPX = _NP + 64
_INDEX = 123
_NC = 2
_NS = 16
_NW = _NC * _NS
_EPW = _E // _NW
_ECH = 2000
_NCHK = _EPW // _ECH
_CAP = 10048
_ACH = 64
_RPT = _NP // _NW
_BR = 512


def _mesh():
    return plsc.VectorSubcoreMesh(core_axis_name="c", subcore_axis_name="s")


def _iota16():
    return lax.iota(jnp.int32, 16)


def _sc_prep(src, dst, ewp):
    @functools.partial(
        pl.kernel,
        out_type=(
            jax.ShapeDtypeStruct((_E,), jnp.float32),
            jax.ShapeDtypeStruct((_NC * _NP,), jnp.float32),
            jax.ShapeDtypeStruct((_NW * _NPX,), jnp.int32),
        ),
        mesh=_mesh(),
        compiler_params=pltpu.CompilerParams(needs_layout_passes=False),
        scratch_types=(
            pltpu.VMEM((_ECH,), jnp.int32),
            pltpu.VMEM((_ECH,), jnp.int32),
            pltpu.VMEM((_ECH,), jnp.float32),
            pltpu.VMEM((_ECH,), jnp.float32),
            pltpu.VMEM((_RPT,), jnp.float32),
            pltpu.VMEM((_NPX,), jnp.int32),
            pltpu.VMEM_SHARED((_NP,), jnp.float32),
        ),
    )
    def k(src_h, dst_h, ewp_h, sig_h, degp_h, needap_h,
          srcb, dstb, ewb, sigb, zf, flagA, spdeg):
        c = lax.axis_index("c")
        s = lax.axis_index("s")
        wid = s * _NC + c
        iota = _iota16()
        ones = jnp.ones((16,), jnp.int32)

        def initz(i, _):
            flagA[pl.ds(i * 16, 16)] = jnp.zeros((16,), jnp.int32)
            return 0
        lax.fori_loop(0, _NPX // 16, initz, 0)

        def initd(i, _):
            zf[pl.ds(i * 16, 16)] = jnp.zeros((16,), jnp.float32)
            return 0
        lax.fori_loop(0, _RPT // 16, initd, 0)
        pltpu.sync_copy(zf, spdeg.at[pl.ds(s * _RPT, _RPT)])
        plsc.subcore_barrier()

        @pl.when(wid == 0)
        def _set_index_flag():
            plsc.store_scatter(
                flagA, [jnp.where(iota == 0, _INDEX, _NP)], ones)

        def chunk(kk, _):
            base = wid * _EPW + kk * _ECH
            pltpu.sync_copy(src_h.at[pl.ds(base, _ECH)], srcb)
            pltpu.sync_copy(dst_h.at[pl.ds(base, _ECH)], dstb)
            pltpu.sync_copy(ewp_h.at[pl.ds(base, _ECH)], ewb)

            def inner(j, _2):
                sl = pl.ds(j * 16, 16)
                s16 = srcb[sl]
                d16 = dstb[sl]
                g16 = 1.0 / (1.0 + jnp.exp(-ewb[sl]))
                sigb[sl] = g16
                plsc.store_scatter(
                    flagA, [jnp.where(d16 == _INDEX, s16, _NP)], ones)
                return 0
            lax.fori_loop(0, _ECH // 16, inner, 0)
            pltpu.sync_copy(sigb, sig_h.at[pl.ds(base, _ECH)])
            pltpu.sync_copy(sigb, spdeg.at[dstb], add=True)
            return 0
        lax.fori_loop(0, _NCHK, chunk, 0)

        pltpu.sync_copy(flagA, needap_h.at[pl.ds(wid * _NPX, _NPX)])
        plsc.subcore_barrier()
        pltpu.sync_copy(spdeg.at[pl.ds(s * _RPT, _RPT)], zf)
        pltpu.sync_copy(zf, degp_h.at[pl.ds(c * _NP + s * _RPT, _RPT)])

    return k(src, dst, ewp)


def _tc_dinv(degp, needap):
    def body(d_ref, n_ref, o_ref, a_ref):
        o_ref[...] = lax.rsqrt(d_ref[0] + d_ref[1] + 1.0)
        a_ref[...] = (jnp.sum(n_ref[...], axis=0) > 0).astype(jnp.int32)

    return pl.pallas_call(
        body,
        grid=(_NP // _BR,),
        in_specs=[
            pl.BlockSpec((_NC, _BR), lambda i: (0, i)),
            pl.BlockSpec((_NW, _BR), lambda i: (0, i)),
        ],
        out_specs=[
            pl.BlockSpec((_BR,), lambda i: (i,)),
            pl.BlockSpec((_BR,), lambda i: (i,)),
        ],
        out_shape=[
            jax.ShapeDtypeStruct((_NP,), jnp.float32),
            jax.ShapeDtypeStruct((_NP,), jnp.int32),
        ],
    )(degp.reshape(_NC, _NP), needap.reshape(_NW, _NPX)[:, :_NP])


def _tc_merge_b(needbp, needa):
    def body(n_ref, a_ref, o_ref):
        m = (jnp.sum(n_ref[...], axis=0) > 0).astype(jnp.int32)
        o_ref[...] = m | a_ref[...]

    return pl.pallas_call(
        body,
        grid=(_NP // _BR,),
        in_specs=[
            pl.BlockSpec((_NW, _BR), lambda i: (0, i)),
            pl.BlockSpec((_BR,), lambda i: (i,)),
        ],
        out_specs=pl.BlockSpec((_BR,), lambda i: (i,)),
        out_shape=jax.ShapeDtypeStruct((_NP,), jnp.int32),
    )(needbp.reshape(_NW, _NPX)[:, :_NP], needa)


def _sc_flag2(src, dst, needa):
    @functools.partial(
        pl.kernel,
        out_type=jax.ShapeDtypeStruct((_NW * _NPX,), jnp.int32),
        mesh=_mesh(),
        compiler_params=pltpu.CompilerParams(needs_layout_passes=False),
        scratch_types=(
            pltpu.VMEM((_NP,), jnp.int32),
            pltpu.VMEM((_NPX,), jnp.int32),
            pltpu.VMEM((_ECH,), jnp.int32),
            pltpu.VMEM((_ECH,), jnp.int32),
        ),
    )
    def k(src_h, dst_h, needa_h, needbp_h, fA, flagB, srcb, dstb):
        c = lax.axis_index("c")
        s = lax.axis_index("s")
        wid = s * _NC + c
        ones = jnp.ones((16,), jnp.int32)
        pltpu.sync_copy(needa_h, fA)

        def initz(i, _):
            flagB[pl.ds(i * 16, 16)] = jnp.zeros((16,), jnp.int32)
            return 0
        lax.fori_loop(0, _NPX // 16, initz, 0)

        def chunk(kk, _):
            base = wid * _EPW + kk * _ECH
            pltpu.sync_copy(src_h.at[pl.ds(base, _ECH)], srcb)
            pltpu.sync_copy(dst_h.at[pl.ds(base, _ECH)], dstb)

            def inner(j, _2):
                sl = pl.ds(j * 16, 16)
                d16 = dstb[sl]
                s16 = srcb[sl]
                f16 = plsc.load_gather(fA, [d16])
                plsc.store_scatter(
                    flagB, [jnp.where(f16 > 0, s16, _NP)], ones)
                return 0
            lax.fori_loop(0, _ECH // 16, inner, 0)
            return 0
        lax.fori_loop(0, _NCHK, chunk, 0)

        pltpu.sync_copy(flagB, needbp_h.at[pl.ds(wid * _NPX, _NPX)])

    return k(src, dst, needa)


def _sc_compact(src, dst, sig, dinv, needa, needb):
    @functools.partial(
        pl.kernel,
        out_type=(
            jax.ShapeDtypeStruct((3 * _NW * _CAP,), jnp.int32),
            jax.ShapeDtypeStruct((3 * _NW * _CAP,), jnp.int32),
            jax.ShapeDtypeStruct((3 * _NW * _CAP,), jnp.float32),
            jax.ShapeDtypeStruct((_NW * 16,), jnp.int32),
        ),
        mesh=_mesh(),
        compiler_params=pltpu.CompilerParams(needs_layout_passes=False),
        scratch_types=(
            pltpu.VMEM((_NP,), jnp.float32),
            pltpu.VMEM((_NP,), jnp.int32),
            pltpu.VMEM((_NP,), jnp.int32),
            pltpu.VMEM((_ECH,), jnp.int32),
            pltpu.VMEM((_ECH,), jnp.int32),
            pltpu.VMEM((_ECH,), jnp.float32),
            pltpu.VMEM((_CAP,), jnp.int32),
            pltpu.VMEM((_CAP,), jnp.int32),
            pltpu.VMEM((_CAP,), jnp.float32),
            pltpu.VMEM((_CAP,), jnp.int32),
            pltpu.VMEM((_CAP,), jnp.int32),
            pltpu.VMEM((_CAP,), jnp.float32),
            pltpu.VMEM((_CAP,), jnp.int32),
            pltpu.VMEM((_CAP,), jnp.int32),
            pltpu.VMEM((_CAP,), jnp.float32),
            pltpu.VMEM((16,), jnp.int32),
        ),
    )
    def k(src_h, dst_h, sig_h, dinv_h, needa_h, needb_h,
          srcc_h, dstc_h, normc_h, cnt_h,
          dinvv, fB, fA, srcb, dstb, sgb,
          s1s, s1d, s1n, s2s, s2d, s2n, s3s, s3d, s3n, cntb):
        c = lax.axis_index("c")
        s = lax.axis_index("s")
        wid = s * _NC + c
        iota = _iota16()

        pltpu.sync_copy(dinv_h, dinvv)
        pltpu.sync_copy(needa_h, fA)
        pltpu.sync_copy(needb_h, fB)

        def zs(i, _):
            sl = pl.ds(i * 16, 16)
            zi = jnp.zeros((16,), jnp.int32)
            zft = jnp.zeros((16,), jnp.float32)
            s1s[sl] = zi; s1d[sl] = zi; s1n[sl] = zft
            s2s[sl] = zi; s2d[sl] = zi; s2n[sl] = zft
            s3s[sl] = zi; s3d[sl] = zi; s3n[sl] = zft
            return 0
        lax.fori_loop(0, _CAP // 16, zs, 0)

        def chunk(kk, carry):
            base = wid * _EPW + kk * _ECH
            pltpu.sync_copy(src_h.at[pl.ds(base, _ECH)], srcb)
            pltpu.sync_copy(dst_h.at[pl.ds(base, _ECH)], dstb)
            pltpu.sync_copy(sig_h.at[pl.ds(base, _ECH)], sgb)

            def inner(j, cr):
                c1, c2, c3 = cr
                sl = pl.ds(j * 16, 16)
                s16 = srcb[sl]
                d16 = dstb[sl]
                g16 = sgb[sl]
                fb16 = plsc.load_gather(fB, [d16])
                fa16 = plsc.load_gather(fA, [d16])
                n16 = plsc.load_gather(dinvv, [s16]) * g16 * \
                    plsc.load_gather(dinvv, [d16])
                m1 = fb16 > 0
                m2m = fa16 > 0
                m3 = d16 == _INDEX
                plsc.store_compressed(s1s.at[pl.ds(c1, 16)], s16, mask=m1)
                plsc.store_compressed(s1d.at[pl.ds(c1, 16)], d16, mask=m1)
                plsc.store_compressed(s1n.at[pl.ds(c1, 16)], n16, mask=m1)
                c1 = c1 + jnp.max(plsc.all_reduce_population_count(m1))
                plsc.store_compressed(s2s.at[pl.ds(c2, 16)], s16, mask=m2m)
                plsc.store_compressed(s2d.at[pl.ds(c2, 16)], d16, mask=m2m)
                plsc.store_compressed(s2n.at[pl.ds(c2, 16)], n16, mask=m2m)
                c2 = c2 + jnp.max(plsc.all_reduce_population_count(m2m))
                plsc.store_compressed(s3s.at[pl.ds(c3, 16)], s16, mask=m3)
                plsc.store_compressed(s3d.at[pl.ds(c3, 16)], d16, mask=m3)
                plsc.store_compressed(s3n.at[pl.ds(c3, 16)], n16, mask=m3)
                c3 = c3 + jnp.max(plsc.all_reduce_population_count(m3))
                return (c1, c2, c3)
            return lax.fori_loop(0, _ECH // 16, inner, carry)

        zero = jnp.zeros((), jnp.int32)
        c1, c2, c3 = lax.fori_loop(0, _NCHK, chunk, (zero, zero, zero))

        @pl.when(wid == 0)
        def _selfloop():
            dv = plsc.load_gather(dinvv, [jnp.where(iota == 0, _INDEX, 0)])
            s3s[pl.ds(c3, 16)] = jnp.where(iota == 0, _INDEX, 0)
            s3d[pl.ds(c3, 16)] = jnp.where(iota == 0, _INDEX, 0)
            s3n[pl.ds(c3, 16)] = jnp.where(iota == 0, dv * dv, 0.0)

        c3 = c3 + jnp.where(wid == 0, 1, 0)

        p1 = ((c1 + _ACH - 1) // _ACH) * _ACH
        p2 = ((c2 + _ACH - 1) // _ACH) * _ACH
        p3 = ((c3 + _ACH - 1) // _ACH) * _ACH
        cntb[pl.ds(0, 16)] = (jnp.where(iota == 0, p1, 0)
                              + jnp.where(iota == 1, p2, 0)
                              + jnp.where(iota == 2, p3, 0))
        pltpu.sync_copy(cntb, cnt_h.at[pl.ds(wid * 16, 16)])

        pltpu.sync_copy(s1s, srcc_h.at[pl.ds((0 * _NW + wid) * _CAP, _CAP)])
        pltpu.sync_copy(s1d, dstc_h.at[pl.ds((0 * _NW + wid) * _CAP, _CAP)])
        pltpu.sync_copy(s1n, normc_h.at[pl.ds((0 * _NW + wid) * _CAP, _CAP)])
        pltpu.sync_copy(s2s, srcc_h.at[pl.ds((1 * _NW + wid) * _CAP, _CAP)])
        pltpu.sync_copy(s2d, dstc_h.at[pl.ds((1 * _NW + wid) * _CAP, _CAP)])
        pltpu.sync_copy(s2n, normc_h.at[pl.ds((1 * _NW + wid) * _CAP, _CAP)])
        pltpu.sync_copy(s3s, srcc_h.at[pl.ds((2 * _NW + wid) * _CAP, _CAP)])
        pltpu.sync_copy(s3d, dstc_h.at[pl.ds((2 * _NW + wid) * _CAP, _CAP)])
        pltpu.sync_copy(s3n, normc_h.at[pl.ds((2 * _NW + wid) * _CAP, _CAP)])

    return k(src, dst, sig, dinv, needa, needb)


def _sc_agg(feat, srcc, dstc, normc, counts, lane):
    @functools.partial(
        pl.kernel,
        out_type=jax.ShapeDtypeStruct((_NC * _NP, 128), jnp.float32),
        mesh=_mesh(),
        compiler_params=pltpu.CompilerParams(needs_layout_passes=False),
        scratch_types=(
            pltpu.VMEM((_ACH,), jnp.int32),
            pltpu.VMEM((_ACH,), jnp.int32),
            pltpu.VMEM((_ACH,), jnp.float32),
            pltpu.VMEM((16,), jnp.int32),
            pltpu.VMEM((_ACH, 128), jnp.float32),
            pltpu.VMEM_SHARED((_NP, 128), jnp.float32),
            pltpu.SemaphoreType.DMA,
        ),
    )
    def k(feat_h, srcc_h, dstc_h, normc_h, cnt_h, aggp_h,
          sbuf, dbuf, nbuf, cntb, rows, spagg, sem):
        c = lax.axis_index("c")
        s = lax.axis_index("s")
        wid = s * _NC + c
        iota = _iota16()

        for i in range(_ACH):
            for q in range(8):
                rows[i, pl.ds(q * 16, 16)] = jnp.zeros((16,), jnp.float32)

        def zs(i, _):
            pltpu.sync_copy(
                rows, spagg.at[pl.ds(s * _RPT + i * _ACH, _ACH), :])
            return 0
        lax.fori_loop(0, _RPT // _ACH, zs, 0)
        plsc.subcore_barrier()

        pltpu.sync_copy(cnt_h.at[pl.ds(wid * 16, 16)], cntb)
        cnt = jnp.max(jnp.where(iota == lane, cntb[pl.ds(0, 16)], 0))
        lbase = (lane * _NW + wid) * _CAP

        def chunk(j, _):
            b = lbase + j * _ACH
            pltpu.sync_copy(srcc_h.at[pl.ds(b, _ACH)], sbuf)
            pltpu.sync_copy(dstc_h.at[pl.ds(b, _ACH)], dbuf)
            pltpu.sync_copy(normc_h.at[pl.ds(b, _ACH)], nbuf)
            pltpu.async_copy(feat_h.at[sbuf], rows, sem).wait()
            for g in range(4):
                n16 = nbuf[pl.ds(g * 16, 16)]
                for i in range(16):
                    sc = jnp.sum(jnp.where(iota == i, n16, 0.0))
                    for q in range(8):
                        sl = pl.ds(q * 16, 16)
                        rows[g * 16 + i, sl] = rows[g * 16 + i, sl] * sc
            pltpu.sync_copy(rows, spagg.at[dbuf], add=True)
            return 0
        lax.fori_loop(0, cnt // _ACH, chunk, 0)
        plsc.subcore_barrier()

        def ex(i, _):
            pltpu.sync_copy(
                spagg.at[pl.ds(s * _RPT + i * _ACH, _ACH), :], rows)
            pltpu.sync_copy(
                rows, aggp_h.at[pl.ds(c * _NP + s * _RPT + i * _ACH, _ACH), :])
            return 0
        lax.fori_loop(0, _RPT // _ACH, ex, 0)

    return k(feat, srcc, dstc, normc, counts)


def _tc_layer(aggp, dinv, feat, W, b):
    def body(a_ref, dv_ref, f_ref, w_ref, b_ref, o_ref):
        dv = dv_ref[...]
        a = a_ref[0] + a_ref[1] + (dv * dv)[:, None] * f_ref[...]
        o_ref[...] = jnp.maximum(
            jnp.dot(a, w_ref[...], preferred_element_type=jnp.float32)
            + b_ref[...], 0.0)

    return pl.pallas_call(
        body,
        grid=(_NP // _BR,),
        in_specs=[
            pl.BlockSpec((_NC, _BR, 128), lambda i: (0, i, 0)),
            pl.BlockSpec((_BR,), lambda i: (i,)),
            pl.BlockSpec((_BR, 128), lambda i: (i, 0)),
            pl.BlockSpec((128, 128), lambda i: (0, 0)),
            pl.BlockSpec((1, 128), lambda i: (0, 0)),
        ],
        out_specs=pl.BlockSpec((_BR, 128), lambda i: (i, 0)),
        out_shape=jax.ShapeDtypeStruct((_NP, 128), jnp.float32),
    )(aggp.reshape(_NC, _NP, 128), dinv, feat, W, b.reshape(1, -1))


def _sc_agg3(feat, srcc, normc, counts):
    @functools.partial(
        pl.kernel,
        out_type=jax.ShapeDtypeStruct((_NC, 128), jnp.float32),
        mesh=_mesh(),
        compiler_params=pltpu.CompilerParams(needs_layout_passes=False),
        scratch_types=(
            pltpu.VMEM((_ACH,), jnp.int32),
            pltpu.VMEM((_ACH,), jnp.float32),
            pltpu.VMEM((16,), jnp.int32),
            pltpu.VMEM((_ACH, 128), jnp.float32),
            pltpu.VMEM((_NS, 128), jnp.float32),
            pltpu.VMEM_SHARED((_NS, 128), jnp.float32),
            pltpu.SemaphoreType.DMA,
        ),
    )
    def k(feat_h, srcc_h, normc_h, cnt_h, out_h,
          sbuf, nbuf, cntb, rows, accb, sp3, sem):
        c = lax.axis_index("c")
        s = lax.axis_index("s")
        wid = s * _NC + c
        iota = _iota16()

        pltpu.sync_copy(cnt_h.at[pl.ds(wid * 16, 16)], cntb)
        cnt = jnp.max(jnp.where(iota == 2, cntb[pl.ds(0, 16)], 0))
        lbase = (2 * _NW + wid) * _CAP

        acc0 = tuple(jnp.zeros((16,), jnp.float32) for _ in range(8))

        def chunk(j, acc):
            b = lbase + j * _ACH
            pltpu.sync_copy(srcc_h.at[pl.ds(b, _ACH)], sbuf)
            pltpu.sync_copy(normc_h.at[pl.ds(b, _ACH)], nbuf)
            pltpu.async_copy(feat_h.at[sbuf], rows, sem).wait()
            acc = list(acc)
            for g in range(4):
                n16 = nbuf[pl.ds(g * 16, 16)]
                for i in range(16):
                    sc = jnp.sum(jnp.where(iota == i, n16, 0.0))
                    for q in range(8):
                        acc[q] = acc[q] + rows[g * 16 + i,
                                               pl.ds(q * 16, 16)] * sc
            return tuple(acc)
        acc = lax.fori_loop(0, cnt // _ACH, chunk, acc0)

        for q in range(8):
            accb[0, pl.ds(q * 16, 16)] = acc[q]
        pltpu.sync_copy(accb.at[pl.ds(0, 1), :], sp3.at[pl.ds(s, 1), :])
        plsc.subcore_barrier()

        @pl.when(s == 0)
        def _reduce():
            pltpu.sync_copy(sp3, accb)
            for q in range(8):
                sl = pl.ds(q * 16, 16)
                v = accb[0, sl]
                for r in range(1, _NS):
                    v = v + accb[r, sl]
                rows[0, sl] = v
            pltpu.sync_copy(rows.at[pl.ds(0, 1), :],
                            out_h.at[pl.ds(c, 1), :])

    return k(feat, srcc, normc, counts)


def _tc_final(agg3, W3p, b3p):
    def body(a_ref, w_ref, b_ref, o_ref):
        a = a_ref[pl.ds(0, 1), :] + a_ref[pl.ds(1, 1), :]
        z = jnp.dot(a, w_ref[...], preferred_element_type=jnp.float32) \
            + b_ref[...]
        m = jnp.max(z)
        lse = jnp.log(jnp.sum(jnp.exp(z - m))) + m
        o_ref[...] = z - lse

    return pl.pallas_call(
        body,
        grid=(1,),
        in_specs=[
            pl.BlockSpec((_NC, 128), lambda i: (0, 0)),
            pl.BlockSpec((128, 128), lambda i: (0, 0)),
            pl.BlockSpec((1, 128), lambda i: (0, 0)),
        ],
        out_specs=pl.BlockSpec((1, 128), lambda i: (0, 0)),
        out_shape=jax.ShapeDtypeStruct((1, 128), jnp.float32),
    )(agg3, W3p, b3p.reshape(1, -1))


def kernel(x, edge_index, edge_weight_params, W1, b1, W2, b2, W3, b3):
    src = edge_index[0].astype(jnp.int32)
    dst = edge_index[1].astype(jnp.int32)
    xp = jnp.pad(x, ((0, _NP - _N), (0, 0)))

    sig, degp, needap = _sc_prep(src, dst, edge_weight_params)
    dinv, needa = _tc_dinv(degp, needap)
    needbp = _sc_flag2(src, dst, needa)
    needb = _tc_merge_b(needbp, needa)
    srcc, dstc, normc, counts = _sc_compact(src, dst, sig, dinv, needa, needb)
    aggp1 = _sc_agg(xp, srcc, dstc, normc, counts, 0)
    h1 = _tc_layer(aggp1, dinv, xp, W1, b1)
    aggp2 = _sc_agg(h1, srcc, dstc, normc, counts, 1)
    h2 = _tc_layer(aggp2, dinv, h1, W2, b2)
    agg3 = _sc_agg3(h2, srcc, normc, counts)
    W3p = jnp.pad(W3, ((0, 0), (0, 128 - 16)))
    b3p = jnp.pad(b3, (0, 128 - 16), constant_values=-1e30)
    out = _tc_final(agg3, W3p, b3p)
    return out[0, :16]
